# Initial kernel scaffold; baseline (speedup 1.0000x reference)
import functools

import jax
import jax.numpy as jnp
from jax import lax
from jax.experimental import pallas as pl
from jax.experimental.pallas import tpu as pltpu
from jax.experimental.pallas import tpu_sc as plsc

_N = 50000
_D = 64
_DH = 32
_E = 800000

_CHUNK = 128
_STEPS = 392
_EPT = _STEPS * _CHUNK
_EPAD = 16 * _EPT
_NROWS = 50048
_RPT = _NROWS // 16

_BLK = 2000
_GRID = _N // _BLK


def _build_segsum():
    mesh = plsc.VectorSubcoreMesh(core_axis_name="c", subcore_axis_name="s")

    @functools.partial(
        pl.kernel,
        out_type=(
            jax.ShapeDtypeStruct((_NROWS, _DH), jnp.float32),
            jax.ShapeDtypeStruct((_NROWS, _DH), jnp.float32),
        ),
        mesh=mesh,
        scratch_types=[
            pltpu.VMEM((_STEPS, _CHUNK), jnp.int32),
            pltpu.VMEM((_STEPS, _CHUNK), jnp.int32),
            pltpu.VMEM((_CHUNK, _DH), jnp.float32),
            pltpu.VMEM((_CHUNK, _DH), jnp.float32),
            pltpu.VMEM_SHARED((_NROWS, _DH), jnp.float32),
            pltpu.SemaphoreType.DMA,
        ],
    )
    def segsum(xlo, xhi, src_hbm, dst_hbm, zeros_hbm, out_lo, out_hi,
               src_v, dst_v, rows_a, rows_b, agg, sem):
        c = lax.axis_index("c")
        s = lax.axis_index("s")

        pltpu.sync_copy(src_hbm.at[pl.ds(s * _STEPS, _STEPS)], src_v)
        pltpu.sync_copy(dst_hbm.at[pl.ds(s * _STEPS, _STEPS)], dst_v)
        pltpu.sync_copy(zeros_hbm, agg.at[pl.ds(s * _RPT, _RPT)])
        plsc.subcore_barrier()

        def run(table):
            def start(g, buf):
                pltpu.make_async_copy(table.at[src_v.at[g]], buf, sem).start()

            def wait(buf):
                pltpu.make_async_copy(table.at[src_v.at[0]], buf, sem).wait()

            def scat(g, buf):
                pltpu.sync_copy(buf, agg.at[dst_v.at[g]], add=True)

            start(0, rows_a)

            def body(k, carry):
                e = 2 * k
                wait(rows_a)
                start(e + 1, rows_b)
                scat(e, rows_a)
                wait(rows_b)

                @pl.when(k < _STEPS // 2 - 1)
                def _():
                    start(e + 2, rows_a)

                scat(e + 1, rows_b)
                return carry

            lax.fori_loop(0, _STEPS // 2, body, 0)

        @pl.when(c == 0)
        def _():
            run(xlo)

        @pl.when(c == 1)
        def _():
            run(xhi)

        plsc.subcore_barrier()

        @pl.when(c == 0)
        def _():
            pltpu.sync_copy(agg.at[pl.ds(s * _RPT, _RPT)],
                            out_lo.at[pl.ds(s * _RPT, _RPT)])

        @pl.when(c == 1)
        def _():
            pltpu.sync_copy(agg.at[pl.ds(s * _RPT, _RPT)],
                            out_hi.at[pl.ds(s * _RPT, _RPT)])

    return segsum


_segsum = _build_segsum()


def _mlp_body(split_out, xlo_r, xhi_r, alo_r, ahi_r, w1_r, b1_r, w2_r, b2_r,
              *outs):
    h = jnp.concatenate(
        [xlo_r[...] + alo_r[...], xhi_r[...] + ahi_r[...]], axis=1)
    t = jnp.tanh(jnp.dot(h, w1_r[...], preferred_element_type=jnp.float32)
                 + b1_r[...])
    o = jnp.dot(t, w2_r[...], preferred_element_type=jnp.float32) + b2_r[...]
    if split_out:
        outs[0][...] = o[:, :_DH]
        outs[1][...] = o[:, _DH:]
    else:
        outs[0][...] = o


def _mlp(xlo, xhi, alo, ahi, w1t, b1, w2t, b2, split_out):
    row_spec = pl.BlockSpec((_BLK, _DH), lambda i: (i, 0))
    full_spec = pl.BlockSpec((_D, _D), lambda i: (0, 0))
    bias_spec = pl.BlockSpec((1, _D), lambda i: (0, 0))
    if split_out:
        out_shape = (jax.ShapeDtypeStruct((_N, _DH), jnp.float32),
                     jax.ShapeDtypeStruct((_N, _DH), jnp.float32))
        out_specs = (row_spec, row_spec)
    else:
        out_shape = jax.ShapeDtypeStruct((_N, _D), jnp.float32)
        out_specs = pl.BlockSpec((_BLK, _D), lambda i: (i, 0))
    return pl.pallas_call(
        functools.partial(_mlp_body, split_out),
        grid=(_GRID,),
        in_specs=[row_spec, row_spec, row_spec, row_spec,
                  full_spec, bias_spec, full_spec, bias_spec],
        out_specs=out_specs,
        out_shape=out_shape,
    )(xlo, xhi, alo, ahi, w1t, b1, w2t, b2)


def _prep_edges(edge_index):
    src = edge_index[0].astype(jnp.int32)
    dst = edge_index[1].astype(jnp.int32)
    pad = _EPAD - _E
    srcp = jnp.concatenate([src, jnp.zeros((pad,), jnp.int32)])
    dstp = jnp.concatenate([dst, jnp.full((pad,), _N, jnp.int32)])
    return (srcp.reshape(_EPAD // _CHUNK, _CHUNK),
            dstp.reshape(_EPAD // _CHUNK, _CHUNK))


def kernel(x, edge_index0, edge_index1, W1_0, b1_0, W2_0, b2_0,
           W1_1, b1_1, W2_1, b2_1):
    x = x.astype(jnp.float32)
    xlo = x[:, :_DH]
    xhi = x[:, _DH:]
    s0, d0 = _prep_edges(edge_index0)
    s1, d1 = _prep_edges(edge_index1)
    zeros = jnp.zeros((_RPT, _DH), jnp.float32)

    a0lo, a0hi = _segsum(xlo, xhi, s0, d0, zeros)
    h1lo, h1hi = _mlp(xlo, xhi, a0lo[:_N], a0hi[:_N],
                      W1_0.T, b1_0.reshape(1, _D), W2_0.T, b2_0.reshape(1, _D),
                      split_out=True)
    a1lo, a1hi = _segsum(h1lo, h1hi, s1, d1, zeros)
    out = _mlp(h1lo, h1hi, a1lo[:_N], a1hi[:_N],
               W1_1.T, b1_1.reshape(1, _D), W2_1.T, b2_1.reshape(1, _D),
               split_out=False)
    return out


# speedup vs baseline: 8.2448x; 8.2448x over previous
import functools

import jax
import jax.numpy as jnp
from jax import lax
from jax.experimental import pallas as pl
from jax.experimental.pallas import tpu as pltpu
from jax.experimental.pallas import tpu_sc as plsc

_N = 50000
_D = 64
_DH = 32
_E = 800000

_CHUNK = 128
_CPB = 4
_BLKE = _CPB * _CHUNK
_NBLK = 98
_EPT = _NBLK * _BLKE
_EPAD = 16 * _EPT
_NROWS = 50048
_RPT = _NROWS // 16

_BLK = 2000
_GRID = _N // _BLK


def _build_segsum():
    mesh = plsc.VectorSubcoreMesh(core_axis_name="c", subcore_axis_name="s")

    @functools.partial(
        pl.kernel,
        out_type=(
            jax.ShapeDtypeStruct((_NROWS, _DH), jnp.float32),
            jax.ShapeDtypeStruct((_NROWS, _DH), jnp.float32),
        ),
        mesh=mesh,
        compiler_params=pltpu.CompilerParams(use_tc_tiling_on_sc=False),
        scratch_types=[
            pltpu.VMEM((2, _CPB, _CHUNK), jnp.int32),
            pltpu.VMEM((2, _CPB, _CHUNK), jnp.int32),
            pltpu.VMEM((_CHUNK, _DH), jnp.float32),
            pltpu.VMEM((_CHUNK, _DH), jnp.float32),
            pltpu.VMEM_SHARED((_NROWS, _DH), jnp.float32),
            pltpu.SemaphoreType.DMA,
            pltpu.SemaphoreType.DMA,
        ],
    )
    def segsum(tlo, thi, src_hbm, dst_hbm, zeros_hbm, out_lo, out_hi,
               src_r, dst_r, rows_a, rows_b, agg, sem_i, sem_g):
        c = lax.axis_index("c")
        s = lax.axis_index("s")
        base = s * _NBLK

        def fetch_idx(b, slot):
            pltpu.make_async_copy(src_hbm.at[pl.ds(base + b, 1)],
                                  src_r.at[pl.ds(slot, 1)], sem_i).start()
            pltpu.make_async_copy(dst_hbm.at[pl.ds(base + b, 1)],
                                  dst_r.at[pl.ds(slot, 1)], sem_i).start()

        def wait_idx(slot):
            pltpu.make_async_copy(src_hbm.at[pl.ds(base, 1)],
                                  src_r.at[pl.ds(slot, 1)], sem_i).wait()
            pltpu.make_async_copy(dst_hbm.at[pl.ds(base, 1)],
                                  dst_r.at[pl.ds(slot, 1)], sem_i).wait()

        fetch_idx(0, 0)
        fetch_idx(1, 1)
        pltpu.sync_copy(zeros_hbm, agg.at[pl.ds(s * _RPT, _RPT)])
        plsc.subcore_barrier()

        def run(table):
            def start_g(idx, buf):
                pltpu.make_async_copy(table.at[idx], buf, sem_g).start()

            def wait_g(buf):
                pltpu.make_async_copy(table.at[src_r.at[0, 0]], buf,
                                      sem_g).wait()

            def do_block(b, slot):
                wait_idx(slot)
                sidx = src_r.at[slot]
                didx = dst_r.at[slot]
                start_g(sidx.at[0], rows_a)
                for j in range(_CPB):
                    buf = rows_a if j % 2 == 0 else rows_b
                    nxt = rows_b if j % 2 == 0 else rows_a
                    wait_g(buf)
                    if j + 1 < _CPB:
                        start_g(sidx.at[j + 1], nxt)
                    pltpu.sync_copy(buf, agg.at[didx.at[j]], add=True)

                @pl.when(b + 2 < _NBLK)
                def _():
                    fetch_idx(b + 2, slot)

            def body(k, carry):
                do_block(2 * k, 0)
                do_block(2 * k + 1, 1)
                return carry

            lax.fori_loop(0, _NBLK // 2, body, 0)

        @pl.when(c == 0)
        def _():
            run(tlo)

        @pl.when(c == 1)
        def _():
            run(thi)

        plsc.subcore_barrier()

        @pl.when(c == 0)
        def _():
            pltpu.sync_copy(agg.at[pl.ds(s * _RPT, _RPT)],
                            out_lo.at[pl.ds(s * _RPT, _RPT)])

        @pl.when(c == 1)
        def _():
            pltpu.sync_copy(agg.at[pl.ds(s * _RPT, _RPT)],
                            out_hi.at[pl.ds(s * _RPT, _RPT)])

    return segsum


_segsum = _build_segsum()


def _mlp_body(split_out, x0, x1, a0, a1, w1_r, b1_r, w2_r, b2_r, *outs):
    w1 = w1_r[...]
    acc = None
    for g, (xr, ar) in enumerate(zip((x0, x1), (a0, a1))):
        hg = xr[...] + ar[...]
        p = jnp.dot(hg, w1[_DH * g:_DH * (g + 1), :],
                    preferred_element_type=jnp.float32)
        acc = p if acc is None else acc + p
    t = jnp.tanh(acc + b1_r[...])
    o = jnp.dot(t, w2_r[...], preferred_element_type=jnp.float32) + b2_r[...]
    if split_out:
        outs[0][...] = o[:, :_DH]
        outs[1][...] = o[:, _DH:]
    else:
        outs[0][...] = o


def _mlp(xg, ag, w1t, b1, w2t, b2, split_out):
    grp_spec = pl.BlockSpec((_BLK, _DH), lambda i: (i, 0))
    full_spec = pl.BlockSpec((_D, _D), lambda i: (0, 0))
    bias_spec = pl.BlockSpec((1, _D), lambda i: (0, 0))
    if split_out:
        out_shape = (jax.ShapeDtypeStruct((_N, _DH), jnp.float32),
                     jax.ShapeDtypeStruct((_N, _DH), jnp.float32))
        out_specs = (grp_spec, grp_spec)
    else:
        out_shape = jax.ShapeDtypeStruct((_N, _D), jnp.float32)
        out_specs = pl.BlockSpec((_BLK, _D), lambda i: (i, 0))
    return pl.pallas_call(
        functools.partial(_mlp_body, split_out),
        grid=(_GRID,),
        in_specs=[grp_spec] * 4 + [full_spec, bias_spec, full_spec, bias_spec],
        out_specs=out_specs,
        out_shape=out_shape,
    )(*xg, *ag, w1t, b1, w2t, b2)


def _prep_edges(edge_index):
    src = edge_index[0].astype(jnp.int32)
    dst = edge_index[1].astype(jnp.int32)
    pad = _EPAD - _E
    srcp = jnp.concatenate([src, jnp.zeros((pad,), jnp.int32)])
    dstp = jnp.concatenate([dst, jnp.full((pad,), _N, jnp.int32)])
    return (srcp.reshape(_EPAD // _BLKE, _CPB, _CHUNK),
            dstp.reshape(_EPAD // _BLKE, _CPB, _CHUNK))


def kernel(x, edge_index0, edge_index1, W1_0, b1_0, W2_0, b2_0,
           W1_1, b1_1, W2_1, b2_1):
    x = x.astype(jnp.float32)
    xg = (x[:, :_DH], x[:, _DH:])
    s0, d0 = _prep_edges(edge_index0)
    s1, d1 = _prep_edges(edge_index1)
    zeros = jnp.zeros((_RPT, _DH), jnp.float32)

    a_l1 = _segsum(xg[0], xg[1], s0, d0, zeros)
    hg = _mlp(xg, a_l1, W1_0.T, b1_0.reshape(1, _D), W2_0.T,
              b2_0.reshape(1, _D), split_out=True)
    a_l2 = _segsum(hg[0], hg[1], s1, d1, zeros)
    out = _mlp(hg, a_l2, W1_1.T, b1_1.reshape(1, _D), W2_1.T,
               b2_1.reshape(1, _D), split_out=False)
    return out


# speedup vs baseline: 11.0542x; 1.3407x over previous
import functools

import jax
import jax.numpy as jnp
from jax import lax
from jax.experimental import pallas as pl
from jax.experimental.pallas import tpu as pltpu
from jax.experimental.pallas import tpu_sc as plsc

_N = 50000
_D = 64
_DH = 32
_E = 800000

_CHUNK = 128
_CPB = 8
_BLKE = _CPB * _CHUNK
_NBLK = 49
_EPT = _NBLK * _BLKE
_EPAD = 16 * _EPT
_NROWS = 50048
_RPT = _NROWS // 16

_BLK = 2000
_GRID = _N // _BLK


def _build_segsum():
    mesh = plsc.VectorSubcoreMesh(core_axis_name="c", subcore_axis_name="s")

    @functools.partial(
        pl.kernel,
        out_type=(
            jax.ShapeDtypeStruct((_NROWS, _DH), jnp.float32),
            jax.ShapeDtypeStruct((_NROWS, _DH), jnp.float32),
        ),
        mesh=mesh,
        compiler_params=pltpu.CompilerParams(use_tc_tiling_on_sc=False),
        scratch_types=[
            pltpu.VMEM((2, _CPB, _CHUNK), jnp.int32),
            pltpu.VMEM((2, _CPB, _CHUNK), jnp.int32),
            pltpu.VMEM((4, _CHUNK, _DH), jnp.float32),
            pltpu.VMEM_SHARED((_NROWS, _DH), jnp.float32),
            pltpu.SemaphoreType.DMA,
            pltpu.SemaphoreType.DMA,
            pltpu.SemaphoreType.DMA,
        ],
    )
    def segsum(tlo, thi, src_hbm, dst_hbm, zeros_hbm, out_lo, out_hi,
               src_r, dst_r, rows_r, agg, sem_i, sem_g, sem_s):
        c = lax.axis_index("c")
        s = lax.axis_index("s")
        base = s * _NBLK

        def fetch_idx(b, slot):
            pltpu.make_async_copy(src_hbm.at[pl.ds(base + b, 1)],
                                  src_r.at[pl.ds(slot, 1)], sem_i).start()
            pltpu.make_async_copy(dst_hbm.at[pl.ds(base + b, 1)],
                                  dst_r.at[pl.ds(slot, 1)], sem_i).start()

        def wait_idx(slot):
            pltpu.make_async_copy(src_hbm.at[pl.ds(base, 1)],
                                  src_r.at[pl.ds(slot, 1)], sem_i).wait()
            pltpu.make_async_copy(dst_hbm.at[pl.ds(base, 1)],
                                  dst_r.at[pl.ds(slot, 1)], sem_i).wait()

        fetch_idx(0, 0)
        fetch_idx(1, 1)
        pltpu.sync_copy(zeros_hbm, agg.at[pl.ds(s * _RPT, _RPT)])
        plsc.subcore_barrier()

        def run(table):
            def start_g(idx, j):
                pltpu.make_async_copy(table.at[idx], rows_r.at[j % 4],
                                      sem_g).start()

            def wait_g(j):
                pltpu.make_async_copy(table.at[src_r.at[0, 0]],
                                      rows_r.at[j % 4], sem_g).wait()

            def start_s(idx, j):
                pltpu.async_copy(rows_r.at[j % 4], agg.at[idx], sem_s,
                                 add=True)

            def wait_s(j):
                pltpu.make_async_copy(rows_r.at[j % 4],
                                      agg.at[dst_r.at[0, 0]], sem_s).wait()

            def do_block(b, slot):
                wait_idx(slot)
                sidx = src_r.at[slot]
                didx = dst_r.at[slot]
                start_g(sidx.at[0], 0)
                start_g(sidx.at[1], 1)
                for j in range(_CPB):
                    wait_g(j)
                    start_s(didx.at[j], j)
                    if j >= 2:
                        wait_s(j - 2)
                    if j + 2 < _CPB:
                        start_g(sidx.at[j + 2], j + 2)
                wait_s(_CPB - 2)
                wait_s(_CPB - 1)

                @pl.when(b + 2 < _NBLK)
                def _():
                    fetch_idx(b + 2, slot)

            def body(k, carry):
                do_block(2 * k, 0)
                do_block(2 * k + 1, 1)
                return carry

            lax.fori_loop(0, (_NBLK - 1) // 2, body, 0)
            do_block(_NBLK - 1, 0)

        @pl.when(c == 0)
        def _():
            run(tlo)

        @pl.when(c == 1)
        def _():
            run(thi)

        plsc.subcore_barrier()

        @pl.when(c == 0)
        def _():
            pltpu.sync_copy(agg.at[pl.ds(s * _RPT, _RPT)],
                            out_lo.at[pl.ds(s * _RPT, _RPT)])

        @pl.when(c == 1)
        def _():
            pltpu.sync_copy(agg.at[pl.ds(s * _RPT, _RPT)],
                            out_hi.at[pl.ds(s * _RPT, _RPT)])

    return segsum


_segsum = _build_segsum()


def _mlp_body(split_out, x0, x1, a0, a1, w1_r, b1_r, w2_r, b2_r, *outs):
    w1 = w1_r[...]
    acc = None
    for g, (xr, ar) in enumerate(zip((x0, x1), (a0, a1))):
        hg = xr[...] + ar[...]
        p = jnp.dot(hg, w1[_DH * g:_DH * (g + 1), :],
                    preferred_element_type=jnp.float32)
        acc = p if acc is None else acc + p
    t = jnp.tanh(acc + b1_r[...])
    o = jnp.dot(t, w2_r[...], preferred_element_type=jnp.float32) + b2_r[...]
    if split_out:
        outs[0][...] = o[:, :_DH]
        outs[1][...] = o[:, _DH:]
    else:
        outs[0][...] = o


def _mlp(xg, ag, w1t, b1, w2t, b2, split_out):
    grp_spec = pl.BlockSpec((_BLK, _DH), lambda i: (i, 0))
    full_spec = pl.BlockSpec((_D, _D), lambda i: (0, 0))
    bias_spec = pl.BlockSpec((1, _D), lambda i: (0, 0))
    if split_out:
        out_shape = (jax.ShapeDtypeStruct((_N, _DH), jnp.float32),
                     jax.ShapeDtypeStruct((_N, _DH), jnp.float32))
        out_specs = (grp_spec, grp_spec)
    else:
        out_shape = jax.ShapeDtypeStruct((_N, _D), jnp.float32)
        out_specs = pl.BlockSpec((_BLK, _D), lambda i: (i, 0))
    return pl.pallas_call(
        functools.partial(_mlp_body, split_out),
        grid=(_GRID,),
        in_specs=[grp_spec] * 4 + [full_spec, bias_spec, full_spec, bias_spec],
        out_specs=out_specs,
        out_shape=out_shape,
    )(*xg, *ag, w1t, b1, w2t, b2)


def _prep_edges(edge_index):
    src = edge_index[0].astype(jnp.int32)
    dst = edge_index[1].astype(jnp.int32)
    pad = _EPAD - _E
    srcp = jnp.concatenate([src, jnp.zeros((pad,), jnp.int32)])
    dstp = jnp.concatenate([dst, jnp.full((pad,), _N, jnp.int32)])
    return (srcp.reshape(_EPAD // _BLKE, _CPB, _CHUNK),
            dstp.reshape(_EPAD // _BLKE, _CPB, _CHUNK))


def kernel(x, edge_index0, edge_index1, W1_0, b1_0, W2_0, b2_0,
           W1_1, b1_1, W2_1, b2_1):
    x = x.astype(jnp.float32)
    xg = (x[:, :_DH], x[:, _DH:])
    s0, d0 = _prep_edges(edge_index0)
    s1, d1 = _prep_edges(edge_index1)
    zeros = jnp.zeros((_RPT, _DH), jnp.float32)

    a_l1 = _segsum(xg[0], xg[1], s0, d0, zeros)
    hg = _mlp(xg, a_l1, W1_0.T, b1_0.reshape(1, _D), W2_0.T,
              b2_0.reshape(1, _D), split_out=True)
    a_l2 = _segsum(hg[0], hg[1], s1, d1, zeros)
    out = _mlp(hg, a_l2, W1_1.T, b1_1.reshape(1, _D), W2_1.T,
               b2_1.reshape(1, _D), split_out=False)
    return out


# speedup vs baseline: 11.5979x; 1.0492x over previous
import functools

import jax
import jax.numpy as jnp
from jax import lax
from jax.experimental import pallas as pl
from jax.experimental.pallas import tpu as pltpu
from jax.experimental.pallas import tpu_sc as plsc

_N = 50000
_D = 64
_DH = 32
_E = 800000

_CHUNK = 128
_CPB = 8
_BLKE = _CPB * _CHUNK
_NBLK = 48
_EPT = _E // 16
_TAIL = _EPT - _NBLK * _BLKE
_TFULL = _TAIL // _CHUNK
_TREM = _TAIL - _TFULL * _CHUNK
_NROWS = 50048
_RPT = _NROWS // 16

_BLK = 2000
_GRID = _N // _BLK


def _build_segsum():
    mesh = plsc.VectorSubcoreMesh(core_axis_name="c", subcore_axis_name="s")

    @functools.partial(
        pl.kernel,
        out_type=(
            jax.ShapeDtypeStruct((_NROWS, _DH), jnp.float32),
            jax.ShapeDtypeStruct((_NROWS, _DH), jnp.float32),
        ),
        mesh=mesh,
        compiler_params=pltpu.CompilerParams(use_tc_tiling_on_sc=False),
        scratch_types=[
            pltpu.VMEM((2 * _BLKE,), jnp.int32),
            pltpu.VMEM((2 * _BLKE,), jnp.int32),
            pltpu.VMEM((_TAIL,), jnp.int32),
            pltpu.VMEM((_TAIL,), jnp.int32),
            pltpu.VMEM((4, _CHUNK, _DH), jnp.float32),
            pltpu.VMEM_SHARED((_NROWS, _DH), jnp.float32),
            pltpu.SemaphoreType.DMA,
            pltpu.SemaphoreType.DMA,
            pltpu.SemaphoreType.DMA,
        ],
    )
    def segsum(tlo, thi, src_hbm, dst_hbm, zeros_hbm, out_lo, out_hi,
               src_r, dst_r, tsrc, tdst, rows_r, agg, sem_i, sem_g, sem_s):
        c = lax.axis_index("c")
        s = lax.axis_index("s")
        base = s * _EPT

        def fetch_idx(b, slot):
            pltpu.make_async_copy(src_hbm.at[pl.ds(base + b * _BLKE, _BLKE)],
                                  src_r.at[pl.ds(slot * _BLKE, _BLKE)],
                                  sem_i).start()
            pltpu.make_async_copy(dst_hbm.at[pl.ds(base + b * _BLKE, _BLKE)],
                                  dst_r.at[pl.ds(slot * _BLKE, _BLKE)],
                                  sem_i).start()

        def wait_idx(slot):
            pltpu.make_async_copy(src_hbm.at[pl.ds(base, _BLKE)],
                                  src_r.at[pl.ds(slot * _BLKE, _BLKE)],
                                  sem_i).wait()
            pltpu.make_async_copy(dst_hbm.at[pl.ds(base, _BLKE)],
                                  dst_r.at[pl.ds(slot * _BLKE, _BLKE)],
                                  sem_i).wait()

        fetch_idx(0, 0)
        fetch_idx(1, 1)
        tail_off = base + _NBLK * _BLKE
        pltpu.sync_copy(src_hbm.at[pl.ds(tail_off, _TAIL)], tsrc)
        pltpu.sync_copy(dst_hbm.at[pl.ds(tail_off, _TAIL)], tdst)
        pltpu.sync_copy(zeros_hbm, agg.at[pl.ds(s * _RPT, _RPT)])
        plsc.subcore_barrier()

        def run(table):
            def buf(j, n=_CHUNK):
                r = rows_r.at[j % 4]
                return r if n == _CHUNK else r.at[pl.ds(0, n)]

            def start_g(idx, j, n=_CHUNK):
                pltpu.make_async_copy(table.at[idx], buf(j, n), sem_g).start()

            def wait_g(j, n=_CHUNK):
                pltpu.make_async_copy(table.at[tsrc.at[pl.ds(0, n)]],
                                      buf(j, n), sem_g).wait()

            def start_s(idx, j, n=_CHUNK):
                pltpu.async_copy(buf(j, n), agg.at[idx], sem_s, add=True)

            def wait_s(j, n=_CHUNK):
                pltpu.make_async_copy(buf(j, n),
                                      agg.at[tdst.at[pl.ds(0, n)]],
                                      sem_s).wait()

            def chunk_idx(r, slot, j):
                return r.at[pl.ds(slot * _BLKE + j * _CHUNK, _CHUNK)]

            def do_block(b, slot):
                wait_idx(slot)
                start_g(chunk_idx(src_r, slot, 0), 0)
                start_g(chunk_idx(src_r, slot, 1), 1)
                for j in range(_CPB):
                    wait_g(j)
                    start_s(chunk_idx(dst_r, slot, j), j)
                    if j >= 2:
                        wait_s(j - 2)
                    if j + 2 < _CPB:
                        start_g(chunk_idx(src_r, slot, j + 2), j + 2)
                wait_s(_CPB - 2)
                wait_s(_CPB - 1)

                @pl.when(b + 2 < _NBLK)
                def _():
                    fetch_idx(b + 2, slot)

            def body(k, carry):
                do_block(2 * k, 0)
                do_block(2 * k + 1, 1)
                return carry

            lax.fori_loop(0, _NBLK // 2, body, 0)

            sizes = [_CHUNK] * _TFULL + [_TREM]
            nt = len(sizes)

            def tidx(r, j, n):
                return r.at[pl.ds(j * _CHUNK, n)]

            start_g(tidx(tsrc, 0, sizes[0]), 0, sizes[0])
            start_g(tidx(tsrc, 1, sizes[1]), 1, sizes[1])
            for j in range(nt):
                wait_g(j, sizes[j])
                start_s(tidx(tdst, j, sizes[j]), j, sizes[j])
                if j >= 2:
                    wait_s(j - 2, sizes[j - 2])
                if j + 2 < nt:
                    start_g(tidx(tsrc, j + 2, sizes[j + 2]), j + 2,
                            sizes[j + 2])
            wait_s(nt - 2, sizes[nt - 2])
            wait_s(nt - 1, sizes[nt - 1])

        @pl.when(c == 0)
        def _():
            run(tlo)

        @pl.when(c == 1)
        def _():
            run(thi)

        plsc.subcore_barrier()

        @pl.when(c == 0)
        def _():
            pltpu.sync_copy(agg.at[pl.ds(s * _RPT, _RPT)],
                            out_lo.at[pl.ds(s * _RPT, _RPT)])

        @pl.when(c == 1)
        def _():
            pltpu.sync_copy(agg.at[pl.ds(s * _RPT, _RPT)],
                            out_hi.at[pl.ds(s * _RPT, _RPT)])

    return segsum


_segsum = _build_segsum()


def _mlp_body(split_out, x0, x1, a0, a1, w1_r, b1_r, w2_r, b2_r, *outs):
    w1 = w1_r[...]
    acc = None
    for g, (xr, ar) in enumerate(zip((x0, x1), (a0, a1))):
        hg = xr[...] + ar[...]
        p = jnp.dot(hg, w1[_DH * g:_DH * (g + 1), :],
                    preferred_element_type=jnp.float32)
        acc = p if acc is None else acc + p
    t = jnp.tanh(acc + b1_r[...])
    o = jnp.dot(t, w2_r[...], preferred_element_type=jnp.float32) + b2_r[...]
    if split_out:
        outs[0][...] = o[:, :_DH]
        outs[1][...] = o[:, _DH:]
    else:
        outs[0][...] = o


def _mlp(xg, ag, w1t, b1, w2t, b2, split_out):
    grp_spec = pl.BlockSpec((_BLK, _DH), lambda i: (i, 0))
    full_spec = pl.BlockSpec((_D, _D), lambda i: (0, 0))
    bias_spec = pl.BlockSpec((1, _D), lambda i: (0, 0))
    if split_out:
        out_shape = (jax.ShapeDtypeStruct((_N, _DH), jnp.float32),
                     jax.ShapeDtypeStruct((_N, _DH), jnp.float32))
        out_specs = (grp_spec, grp_spec)
    else:
        out_shape = jax.ShapeDtypeStruct((_N, _D), jnp.float32)
        out_specs = pl.BlockSpec((_BLK, _D), lambda i: (i, 0))
    return pl.pallas_call(
        functools.partial(_mlp_body, split_out),
        grid=(_GRID,),
        in_specs=[grp_spec] * 4 + [full_spec, bias_spec, full_spec, bias_spec],
        out_specs=out_specs,
        out_shape=out_shape,
    )(*xg, *ag, w1t, b1, w2t, b2)


def _prep_edges(edge_index):
    return (edge_index[0].astype(jnp.int32), edge_index[1].astype(jnp.int32))


def kernel(x, edge_index0, edge_index1, W1_0, b1_0, W2_0, b2_0,
           W1_1, b1_1, W2_1, b2_1):
    x = x.astype(jnp.float32)
    xg = (x[:, :_DH], x[:, _DH:])
    s0, d0 = _prep_edges(edge_index0)
    s1, d1 = _prep_edges(edge_index1)
    zeros = jnp.zeros((_RPT, _DH), jnp.float32)

    a_l1 = _segsum(xg[0], xg[1], s0, d0, zeros)
    hg = _mlp(xg, a_l1, W1_0.T, b1_0.reshape(1, _D), W2_0.T,
              b2_0.reshape(1, _D), split_out=True)
    a_l2 = _segsum(hg[0], hg[1], s1, d1, zeros)
    out = _mlp(hg, a_l2, W1_1.T, b1_1.reshape(1, _D), W2_1.T,
               b2_1.reshape(1, _D), split_out=False)
    return out
